# baseline (speedup 1.0000x reference)
import functools

import jax
import jax.numpy as jnp
from jax.experimental import pallas as pl
from jax.experimental.pallas import tpu as pltpu
from jax.experimental.pallas import tpu_sc as plsc

_NUM_STEPS = 3
_TOPK = 8
_TR = 512
_W = 16
_DEPTH = 4

_NW = 32
_PER_W = 256
_CH = 64
_NCH = _PER_W // _CH


def _f32dot(a, b):
    return jax.lax.dot_general(a, b, (((1,), (0,)), ((), ())),
                               preferred_element_type=jnp.float32)


def _norm_keys(st, Wnp, bnp):
    keys = _f32dot(st, Wnp) + bnp
    nrm = jnp.sqrt(jnp.sum(keys * keys, axis=1, keepdims=True))
    return keys / jnp.maximum(nrm, 1e-12)


def _topk_tile(kn, knt, n, base):
    tcw = knt.shape[0]
    c = n // _W
    neg = jnp.float32(-jnp.inf)
    simt = jax.lax.dot_general(kn, knt, (((1,), (1,)), ((), ())),
                               preferred_element_type=jnp.float32)
    r_iota = jax.lax.broadcasted_iota(jnp.int32, (n, tcw), 0)
    o_iota = jax.lax.broadcasted_iota(jnp.int32, (n, tcw), 1)
    z = jnp.where(r_iota == o_iota + base, neg, simt)
    z3 = z.reshape(c, _W, tcw)
    iw3 = jax.lax.broadcasted_iota(jnp.int32, (c, _W, tcw), 1)
    Ms, Ps = [], []
    for l in range(_DEPTH):
        ml = jnp.max(z3, axis=1)
        tl = jnp.min(jnp.where(z3 == ml[:, None, :], iw3, _W), axis=1)
        Ms.append(ml)
        Ps.append(tl)
        if l + 1 < _DEPTH:
            z3 = jnp.where(iw3 == tl[:, None, :], neg, z3)

    c_iota = jax.lax.broadcasted_iota(jnp.int32, (c, tcw), 0)
    lvl = jnp.zeros((c, tcw), jnp.int32)
    hv = Ms[0]
    hp = Ps[0]
    self_row = jax.lax.broadcasted_iota(jnp.int32, (1, tcw), 1) + base
    rows = [self_row]
    for _j in range(_TOPK - 1):
        m = jnp.max(hv, axis=0, keepdims=True)
        cpick = jnp.min(jnp.where(hv == m, c_iota, c), axis=0,
                        keepdims=True)
        pick = c_iota == cpick
        pos = jnp.min(jnp.where(pick, hp, _W), axis=0, keepdims=True)
        rows.append(cpick * _W + pos)
        lvl = lvl + pick.astype(jnp.int32)
        nxt_v = jnp.where(lvl == 1, Ms[1],
                          jnp.where(lvl == 2, Ms[2],
                                    jnp.where(lvl == 3, Ms[3], neg)))
        nxt_p = jnp.where(lvl == 1, Ps[1],
                          jnp.where(lvl == 2, Ps[2], Ps[3]))
        hv = jnp.where(pick, nxt_v, hv)
        hp = jnp.where(pick, nxt_p, hp)
    return jnp.concatenate(rows, axis=0)


def _mlp_update(stt, nsum_t, Wnp, bnp, W1a, W1b, b1, gamma, beta, W2, b2):
    agg = _f32dot(nsum_t * (1.0 / _TOPK), Wnp) + bnp
    h = _f32dot(stt, W1a) + _f32dot(agg, W1b) + b1
    mu = jnp.mean(h, axis=1, keepdims=True)
    var = jnp.mean((h - mu) ** 2, axis=1, keepdims=True)
    hn = (h - mu) * jax.lax.rsqrt(var + 1e-5) * gamma + beta
    a = hn * (1.0 / (1.0 + jnp.exp(-hn)))
    return stt + _f32dot(a, W2) + b2


def _idx_body(x_ref, Wnp_ref, bnp_ref, idx_ref):
    b = pl.program_id(0)
    n = x_ref.shape[1]
    st = x_ref[0]
    kn = _norm_keys(st, Wnp_ref[...], bnp_ref[...])
    for ct in range(n // _TR):
        knt = kn[ct * _TR:(ct + 1) * _TR]
        idx_ref[0, :, pl.ds(ct * _TR, _TR)] = (
            _topk_tile(kn, knt, n, ct * _TR) + b * n)


def _update_idx_body(x_ref, nsum_ref, Wnp_ref, bnp_ref, W1a_ref, W1b_ref,
                     b1_ref, gamma_ref, beta_ref, W2_ref, b2_ref,
                     newstate_ref, idx_ref):
    b = pl.program_id(0)
    n = x_ref.shape[1]
    st = x_ref[0]
    ns = nsum_ref[0]
    tiles = []
    for rt in range(n // _TR):
        sl = slice(rt * _TR, (rt + 1) * _TR)
        tiles.append(_mlp_update(st[sl], ns[sl], Wnp_ref[...], bnp_ref[...],
                                 W1a_ref[...], W1b_ref[...], b1_ref[...],
                                 gamma_ref[...], beta_ref[...], W2_ref[...],
                                 b2_ref[...]))
    newst = jnp.concatenate(tiles, axis=0)
    newstate_ref[0] = newst
    kn = _norm_keys(newst, Wnp_ref[...], bnp_ref[...])
    for ct in range(n // _TR):
        knt = kn[ct * _TR:(ct + 1) * _TR]
        idx_ref[0, :, pl.ds(ct * _TR, _TR)] = (
            _topk_tile(kn, knt, n, ct * _TR) + b * n)


def _update_readout_body(x_ref, nsum_ref, Wnp_ref, bnp_ref, W1a_ref, W1b_ref,
                         b1_ref, gamma_ref, beta_ref, W2_ref, b2_ref,
                         Wo_ref, bo_ref, out_ref):
    n = x_ref.shape[1]
    st = x_ref[0]
    ns = nsum_ref[0]
    acc = jnp.zeros((1, st.shape[1]), jnp.float32)
    for rt in range(n // _TR):
        sl = slice(rt * _TR, (rt + 1) * _TR)
        newt = _mlp_update(st[sl], ns[sl], Wnp_ref[...], bnp_ref[...],
                           W1a_ref[...], W1b_ref[...], b1_ref[...],
                           gamma_ref[...], beta_ref[...], W2_ref[...],
                           b2_ref[...])
        acc = acc + jnp.sum(newt, axis=0, keepdims=True)
    out_ref[0] = _f32dot(acc * (1.0 / n), Wo_ref[...]) + bo_ref[...]


def _sc_gather_body(table_hbm, gidx_hbm, out_hbm, idx_v, selfr_v, rows_v,
                    obuf_v, *sems):
    k = _TOPK
    d = table_hbm.shape[1]
    n = gidx_hbm.shape[2]
    c = jax.lax.axis_index("c")
    s = jax.lax.axis_index("s")
    wid = s * 2 + c
    b = wid // (n // _PER_W)
    n0 = (wid % (n // _PER_W)) * _PER_W
    base = wid * _PER_W
    for j in range(1, k):
        pltpu.sync_copy(gidx_hbm.at[b, j, pl.ds(n0, _PER_W)],
                        idx_v.at[j - 1])
    copies = [None] * _NCH

    def fire(ch):
        slot = ch % 2
        cps = [pltpu.async_copy(table_hbm.at[pl.ds(base + ch * _CH, _CH)],
                                selfr_v.at[slot], sems[slot])]
        for j in range(k - 1):
            cps.append(pltpu.async_copy(
                table_hbm.at[idx_v.at[j, pl.ds(ch * _CH, _CH)]],
                rows_v.at[slot, j], sems[slot]))
        copies[ch] = cps

    fire(0)
    if _NCH > 1:
        fire(1)
    for ch in range(_NCH):
        slot = ch % 2
        for cp in copies[ch]:
            cp.wait()

        def body(r, carry):
            for col in range(d // 16):
                sl = pl.ds(col * 16, 16)
                v = selfr_v[slot, r, sl]
                for j in range(k - 1):
                    v = v + rows_v[slot, j, r, sl]
                obuf_v[r, sl] = v
            return carry

        jax.lax.fori_loop(0, _CH, body, 0)
        pltpu.sync_copy(obuf_v, out_hbm.at[pl.ds(base + ch * _CH, _CH)])
        if ch + 2 < _NCH:
            fire(ch + 2)


def _sc_gather_sum(table, gidx):
    rows, d = table.shape
    mesh = plsc.VectorSubcoreMesh(core_axis_name="c", subcore_axis_name="s")
    return pl.kernel(
        _sc_gather_body,
        mesh=mesh,
        compiler_params=pltpu.CompilerParams(use_tc_tiling_on_sc=False),
        out_type=jax.ShapeDtypeStruct((rows, d), jnp.float32),
        scratch_types=[
            pltpu.VMEM((_TOPK - 1, _PER_W), jnp.int32),
            pltpu.VMEM((2, _CH, d), jnp.float32),
            pltpu.VMEM((2, _TOPK - 1, _CH, d), jnp.float32),
            pltpu.VMEM((_CH, d), jnp.float32),
        ] + [pltpu.SemaphoreType.DMA] * 2,
    )(table, gidx)


@jax.jit
def kernel(x, W_np, b_np, W1, b1, gamma, beta, W2, b2, Wo, bo):
    B, N, D = x.shape
    H = W_np.shape[1]
    O = Wo.shape[1]
    W1a = W1[:D]
    W1b = W1[D:]
    row = lambda v: v.reshape(1, -1)
    bnp, b1r, gr, br, b2r, bor = (row(b_np), row(b1), row(gamma), row(beta),
                                  row(b2), row(bo))

    full = lambda shape: pl.BlockSpec(shape, lambda b: (0,) * len(shape))
    bspec = lambda shape: pl.BlockSpec((1,) + shape,
                                       lambda b: (b,) + (0,) * len(shape))
    wspecs = [full((D, H)), full((1, H)), full((D, H)), full((H, H)),
              full((1, H)), full((1, H)), full((1, H)), full((H, D)),
              full((1, D))]

    idx_call = pl.pallas_call(
        _idx_body, grid=(B,),
        in_specs=[bspec((N, D)), full((D, H)), full((1, H))],
        out_specs=bspec((_TOPK, N)),
        out_shape=jax.ShapeDtypeStruct((B, _TOPK, N), jnp.int32),
    )
    upd_idx_call = pl.pallas_call(
        _update_idx_body, grid=(B,),
        in_specs=[bspec((N, D)), bspec((N, D))] + wspecs,
        out_specs=[bspec((N, D)), bspec((_TOPK, N))],
        out_shape=[jax.ShapeDtypeStruct((B, N, D), jnp.float32),
                   jax.ShapeDtypeStruct((B, _TOPK, N), jnp.int32)],
    )
    upd_out_call = pl.pallas_call(
        _update_readout_body, grid=(B,),
        in_specs=[bspec((N, D)), bspec((N, D))] + wspecs +
                 [full((D, O)), full((1, O))],
        out_specs=bspec((1, O)),
        out_shape=jax.ShapeDtypeStruct((B, 1, O), jnp.float32),
    )

    def gather(state, idx):
        nsum = _sc_gather_sum(state.reshape(B * N, D), idx)
        return nsum.reshape(B, N, D)

    state = x
    idx = idx_call(x, W_np, bnp)
    for _step in range(_NUM_STEPS - 1):
        nsum = gather(state, idx)
        state, idx = upd_idx_call(state, nsum, W_np, bnp, W1a, W1b, b1r,
                                  gr, br, W2, b2r)
    nsum = gather(state, idx)
    out = upd_out_call(state, nsum, W_np, bnp, W1a, W1b, b1r, gr, br, W2,
                       b2r, Wo, bor)
    return out.reshape(B, O)
